# baseline (speedup 1.0000x reference)
import functools

import jax
import jax.numpy as jnp
from jax import lax
from jax.experimental import pallas as pl
from jax.experimental.pallas import tpu as pltpu
from jax.experimental.pallas import tpu_sc as plsc

_ROWS = 16384
_COLS = 512

_RT = 16384

_TBLK = 256
_TGRID = _RT // _TBLK


def _tc_body(p_ref, t_ref, o_ref):
    def strip(i, acc):
        x = p_ref[pl.ds(i * 8, 8), :]
        t = t_ref[pl.ds(i * 8, 8), :]
        a = jnp.abs(x)
        sp = jnp.log(1.0 + jnp.exp(-a))
        loss = 0.5 * a - 0.125 * (x * t) + sp
        return acc + jnp.where(t > 0.0, loss, 0.0)

    acc = lax.fori_loop(0, _TBLK // 8, strip,
                        jnp.zeros((8, _COLS), jnp.float32))

    @pl.when(pl.program_id(0) == 0)
    def _init():
        o_ref[0] = 0.0

    o_ref[0] += jnp.sum(acc)


def _tc_call(p2, t2):
    return pl.pallas_call(
        _tc_body,
        grid=(_TGRID,),
        in_specs=[
            pl.BlockSpec((_TBLK, _COLS), lambda i: (i, 0)),
            pl.BlockSpec((_TBLK, _COLS), lambda i: (i, 0)),
        ],
        out_specs=pl.BlockSpec(memory_space=pltpu.SMEM),
        out_shape=jax.ShapeDtypeStruct((1,), jnp.float32),
    )(p2, t2)


_NW = 32
_SCROWS = _ROWS - _RT
_RPW = _SCROWS // _NW
_CHR = 16
_NCH = _RPW // _CHR
_VEC = 16
_CPV = _COLS // _VEC

_P4 = (1.4158395336e-04, 9.9542662419e-01, -4.6407059668e-01,
       2.1640848063e-01, -5.4862281195e-02)


def _bce_vec(x, t):
    a = jnp.abs(x)
    u = jnp.exp(-a)
    p = _P4[4]
    for c in (_P4[3], _P4[2], _P4[1], _P4[0]):
        p = p * u + c
    loss = 0.5 * a - 0.125 * (x * t) + p
    return jnp.where(t > 0.0, loss, 0.0)


def _sc_call(p2, t2):
    mesh = plsc.VectorSubcoreMesh(core_axis_name="c", subcore_axis_name="s")

    @functools.partial(
        pl.kernel,
        mesh=mesh,
        out_type=jax.ShapeDtypeStruct((_NW, _VEC), jnp.float32),
        scratch_types=[
            pltpu.VMEM((2, _CHR, _COLS), jnp.float32),
            pltpu.VMEM((2, _CHR, _COLS), jnp.float32),
            pltpu.VMEM((_VEC,), jnp.float32),
            pltpu.SemaphoreType.DMA((2,)),
            pltpu.SemaphoreType.DMA((2,)),
        ],
    )
    def sck(p_hbm, t_hbm, out_hbm, pbuf, tbuf, accv, psem, tsem):
        wid = lax.axis_index("s") * 2 + lax.axis_index("c")
        row0 = _RT + wid * _RPW

        def p_copy(ci, slot):
            r0 = row0 + ci * _CHR
            return pltpu.make_async_copy(
                p_hbm.at[pl.ds(r0, _CHR), :], pbuf.at[slot], psem.at[slot])

        def t_copy(ci, slot):
            r0 = row0 + ci * _CHR
            return pltpu.make_async_copy(
                t_hbm.at[pl.ds(r0, _CHR), :], tbuf.at[slot], tsem.at[slot])

        p_copy(0, 0).start()
        t_copy(0, 0).start()

        z = jnp.zeros((_VEC,), jnp.float32)
        accs = (z, z)
        for ci in range(_NCH):
            slot = ci % 2
            nxt = (ci + 1) % 2
            if ci + 1 < _NCH:
                p_copy(ci + 1, nxt).start()
                t_copy(ci + 1, nxt).start()
            p_copy(ci, slot).wait()
            t_copy(ci, slot).wait()

            pb = pbuf.at[slot]
            tb = tbuf.at[slot]

            def row_body(r, accs, pb=pb, tb=tb):
                def grp_body(g, accs):
                    accs = list(accs)
                    base = g * (8 * _VEC)
                    for c in range(8):
                        x = pb[r, pl.ds(base + c * _VEC, _VEC)]
                        t = tb[r, pl.ds(base + c * _VEC, _VEC)]
                        accs[c % 2] = accs[c % 2] + _bce_vec(x, t)
                    return tuple(accs)

                return lax.fori_loop(0, _CPV // 8, grp_body, accs)

            accs = lax.fori_loop(0, _CHR, row_body, accs)

        accv[...] = accs[0] + accs[1]
        pltpu.sync_copy(accv, out_hbm.at[wid])

    return sck(p2, t2)


def kernel(pred_map, target_map):
    p = pred_map.reshape(_ROWS, _COLS)
    t = target_map.reshape(_ROWS, _COLS)
    out_tc = _tc_call(p, t)
    return out_tc[0]


# speedup vs baseline: 2.0321x; 2.0321x over previous
import jax
import jax.numpy as jnp
from jax.experimental import pallas as pl
from jax.experimental.pallas import tpu as pltpu

_ROWS = 16384
_COLS = 512
_TBLK = 1024
_TGRID = _ROWS // _TBLK


def _tc_body(p_ref, t_ref, o_ref):
    x = p_ref[...]
    t = t_ref[...]
    a = jnp.abs(x)
    sp = jnp.log(1.0 + jnp.exp(-a))
    loss = 0.5 * a - 0.125 * (x * t) + sp
    loss = jnp.where(t > 0.0, loss, 0.0)
    part = jnp.sum(loss)

    @pl.when(pl.program_id(0) == 0)
    def _init():
        o_ref[0] = 0.0

    o_ref[0] += part


def kernel(pred_map, target_map):
    p = pred_map.reshape(_ROWS, _COLS)
    t = target_map.reshape(_ROWS, _COLS)
    out = pl.pallas_call(
        _tc_body,
        grid=(_TGRID,),
        in_specs=[
            pl.BlockSpec((_TBLK, _COLS), lambda i: (i, 0)),
            pl.BlockSpec((_TBLK, _COLS), lambda i: (i, 0)),
        ],
        out_specs=pl.BlockSpec(memory_space=pltpu.SMEM),
        out_shape=jax.ShapeDtypeStruct((1,), jnp.float32),
    )(p, t)
    return out[0]


# speedup vs baseline: 2.1950x; 1.0802x over previous
import jax
import jax.numpy as jnp
from jax.experimental import pallas as pl
from jax.experimental.pallas import tpu as pltpu

_ROWS = 16384
_COLS = 512
_TBLK = 2048
_TGRID = _ROWS // _TBLK


def _tc_body(p_ref, t_ref, o_ref):
    x = p_ref[...]
    t = t_ref[...]
    a = jnp.abs(x)
    sp = jnp.log(1.0 + jnp.exp(-a))
    loss = 0.5 * a - 0.125 * (x * t) + sp
    loss = jnp.where(t > 0.0, loss, 0.0)
    part = jnp.sum(loss)

    @pl.when(pl.program_id(0) == 0)
    def _init():
        o_ref[0] = 0.0

    o_ref[0] += part


def kernel(pred_map, target_map):
    p = pred_map.reshape(_ROWS, _COLS)
    t = target_map.reshape(_ROWS, _COLS)
    out = pl.pallas_call(
        _tc_body,
        grid=(_TGRID,),
        in_specs=[
            pl.BlockSpec((_TBLK, _COLS), lambda i: (i, 0)),
            pl.BlockSpec((_TBLK, _COLS), lambda i: (i, 0)),
        ],
        out_specs=pl.BlockSpec(memory_space=pltpu.SMEM),
        out_shape=jax.ShapeDtypeStruct((1,), jnp.float32),
    )(p, t)
    return out[0]
